# baseline (speedup 1.0000x reference)
import functools
import math

import jax
import jax.numpy as jnp
from jax import lax
from jax.experimental import pallas as pl
from jax.experimental.pallas import tpu as pltpu
from jax.experimental.pallas import tpu_sc as plsc

VOCAB = 1000000
EMB = 64
NC = 2
NS = 16
NW = NC * NS
CHUNK = 128
SCALE = math.sqrt(EMB)


def _emb_kernel(n_chunks, table_hbm, idx_hbm, out_hbm,
                idx_v, in0, in1, out0, out1, gsem0, gsem1, osem0, osem1):
    wid = lax.axis_index("s") * NC + lax.axis_index("c")
    base = wid * (n_chunks * CHUNK)

    pltpu.sync_copy(idx_hbm.at[wid], idx_v)

    pltpu.async_copy(table_hbm.at[idx_v.at[0]], in0, gsem0)
    pltpu.async_copy(table_hbm.at[idx_v.at[1]], in1, gsem1)

    def do_chunk(i, buf_in, buf_out, gsem, osem):
        pltpu.make_async_copy(table_hbm.at[idx_v.at[i]], buf_in, gsem).wait()

        @pl.when(i >= 2)
        def _():
            pltpu.make_async_copy(
                buf_out, out_hbm.at[pl.ds(base + (i - 2) * CHUNK, CHUNK)], osem
            ).wait()

        def scale_row(r):
            for c in range(EMB // 16):
                sl = pl.ds(c * 16, 16)
                buf_out[r, sl] = buf_in[r, sl] * SCALE

        pl.loop(0, CHUNK)(scale_row)

        pltpu.async_copy(
            buf_out, out_hbm.at[pl.ds(base + i * CHUNK, CHUNK)], osem)

        @pl.when(i + 2 < n_chunks)
        def _():
            pltpu.async_copy(table_hbm.at[idx_v.at[i + 2]], buf_in, gsem)

    def body(j):
        do_chunk(j, in0, out0, gsem0, osem0)
        do_chunk(j + 1, in1, out1, gsem1, osem1)

    pl.loop(0, n_chunks, step=2)(body)

    pltpu.make_async_copy(
        out0, out_hbm.at[pl.ds(base + (n_chunks - 2) * CHUNK, CHUNK)], osem0
    ).wait()
    pltpu.make_async_copy(
        out1, out_hbm.at[pl.ds(base + (n_chunks - 1) * CHUNK, CHUNK)], osem1
    ).wait()


def kernel(tokens, table):
    orig_shape = tokens.shape
    n_tok = tokens.shape[0] * tokens.shape[1]
    assert n_tok % (NW * CHUNK) == 0
    n_chunks = n_tok // (NW * CHUNK)
    idx = jnp.reshape(tokens.astype(jnp.int32), (NW, n_chunks, CHUNK))

    mesh = plsc.VectorSubcoreMesh(
        core_axis_name="c", subcore_axis_name="s",
        num_cores=NC, num_subcores=NS)

    run = functools.partial(
        pl.kernel,
        out_type=jax.ShapeDtypeStruct((n_tok, EMB), jnp.float32),
        mesh=mesh,
        compiler_params=pltpu.CompilerParams(use_tc_tiling_on_sc=False),
        scratch_types=[
            pltpu.VMEM((n_chunks, CHUNK), jnp.int32),
            pltpu.VMEM((CHUNK, EMB), jnp.float32),
            pltpu.VMEM((CHUNK, EMB), jnp.float32),
            pltpu.VMEM((CHUNK, EMB), jnp.float32),
            pltpu.VMEM((CHUNK, EMB), jnp.float32),
            pltpu.SemaphoreType.DMA,
            pltpu.SemaphoreType.DMA,
            pltpu.SemaphoreType.DMA,
            pltpu.SemaphoreType.DMA,
        ],
    )(functools.partial(_emb_kernel, n_chunks))

    out = run(table, idx)
    return out.reshape(orig_shape[0], orig_shape[1], EMB)
